# baseline (speedup 1.0000x reference)
import jax
import jax.numpy as jnp
from jax import lax
from jax.experimental import pallas as pl
from jax.experimental.pallas import tpu as pltpu
from jax.experimental.pallas import tpu_sc as plsc

B = 16
L = 2048
DIM = 768
HALF = DIM // 2
NC = 2
NS = 16
LANES = 16
K = 32
CK = 40
G = HALF // LANES


def _sc_body(ids_hbm, mask_hbm, tab_hbm, out_hbm, idx_v, mask_v, rows_a,
             rows_b, corr_v, acc_v, sem):
    c = lax.axis_index("c")
    s = lax.axis_index("s")
    w = s * NC + c
    i = w // 2
    h = w % 2

    ids_cp = pltpu.make_async_copy(ids_hbm.at[i], idx_v.at[pl.ds(0, L)], sem)
    ids_cp.start()
    pltpu.sync_copy(mask_hbm.at[i], mask_v)
    ids_cp.wait()

    for g in range(G):
        acc_v[pl.ds(g * LANES, LANES)] = jnp.zeros((LANES,), jnp.float32)

    def prep_body(t, cnt):
        m = mask_v[pl.ds(t * LANES, LANES)]
        v = idx_v[pl.ds(t * LANES, LANES)]
        idx_v[pl.ds(t * LANES, LANES)] = v * 2 + h
        return cnt + jnp.where(m == 1, 1, 0).astype(jnp.int32)

    cnt = lax.fori_loop(0, L // LANES, prep_body,
                        jnp.zeros((LANES,), jnp.int32), unroll=4)
    n = cnt[0]
    for t in range(1, LANES):
        n = n + cnt[t]
    for t in range(CK // 8):
        idx_v[pl.ds(L + t * 8, 8)] = jnp.zeros((8,), jnp.int32)

    T = (n + K - 1) // K

    def make_copy(j, buf):
        return pltpu.make_async_copy(
            tab_hbm.at[idx_v.at[pl.ds(j * K, K)]], buf, sem)

    def accum(buf):
        for g in range(G):
            gs = pl.ds(g * LANES, LANES)
            p = [buf[q, gs] for q in range(4)]
            for r in range(4, K, 4):
                for q in range(4):
                    p[q] = p[q] + buf[r + q, gs]
            plsc.addupdate(acc_v.at[gs], (p[0] + p[1]) + (p[2] + p[3]))

    @pl.when(T > 0)
    def _():
        make_copy(0, rows_a).start()

    def loop_body(j, _):
        even = lax.rem(j, 2) == 0
        nxt = j + 1 < T

        @pl.when(jnp.logical_and(nxt, even))
        def _():
            make_copy(j + 1, rows_b).start()

        @pl.when(jnp.logical_and(nxt, jnp.logical_not(even)))
        def _():
            make_copy(j + 1, rows_a).start()

        @pl.when(even)
        def _():
            make_copy(j, rows_a).wait()
            accum(rows_a)

        @pl.when(jnp.logical_not(even))
        def _():
            make_copy(j, rows_b).wait()
            accum(rows_b)

        return 0

    lax.fori_loop(0, T, loop_body, 0)

    @pl.when(T > 0)
    def _():
        start = (n // 8) * 8
        corr_cp = pltpu.make_async_copy(
            tab_hbm.at[idx_v.at[pl.ds(start, CK)]], corr_v, sem)
        corr_cp.start()
        corr_cp.wait()
        top = T * K
        for r in range(CK):
            pos = start + r
            sub = jnp.logical_and(pos >= n, pos < top)
            wgt = jnp.where(sub, -1.0, 0.0).astype(jnp.float32)
            for g in range(G):
                gs = pl.ds(g * LANES, LANES)
                plsc.addupdate(acc_v.at[gs], corr_v[r, gs] * wgt)

    nf = n.astype(jnp.float32)
    for g in range(G):
        acc_v[pl.ds(g * LANES, LANES)] = acc_v[pl.ds(g * LANES, LANES)] / nf
    pltpu.sync_copy(acc_v, out_hbm.at[w])


@jax.jit
def _sc_call(ids, mask, tab):
    mesh = plsc.VectorSubcoreMesh(core_axis_name="c", subcore_axis_name="s",
                                  num_cores=NC, num_subcores=NS)
    fn = pl.kernel(
        _sc_body,
        out_type=jax.ShapeDtypeStruct((NC * NS, HALF), jnp.float32),
        mesh=mesh,
        scratch_types=[
            pltpu.VMEM((L + CK,), jnp.int32),
            pltpu.VMEM((L,), jnp.int32),
            pltpu.VMEM((K, HALF), jnp.float32),
            pltpu.VMEM((K, HALF), jnp.float32),
            pltpu.VMEM((CK, HALF), jnp.float32),
            pltpu.VMEM((HALF,), jnp.float32),
            pltpu.SemaphoreType.DMA,
        ],
    )
    return fn(ids, mask, tab)


def kernel(tag_input_ids, tag_attention_mask, table):
    ids = tag_input_ids.astype(jnp.int32)
    mask = tag_attention_mask.astype(jnp.int32)
    tab = table.reshape(2 * table.shape[0], HALF)
    out2 = _sc_call(ids, mask, tab)
    return out2.reshape(B, DIM)


# speedup vs baseline: 4.0963x; 4.0963x over previous
import jax
import jax.numpy as jnp
from jax import lax
from jax.experimental import pallas as pl
from jax.experimental.pallas import tpu as pltpu
from jax.experimental.pallas import tpu_sc as plsc

B = 16
L = 2048
DIM = 768
HALF = DIM // 2
NC = 2
NS = 16
LANES = 16
K = 32
CK = 40
G = HALF // LANES


def _sc_body(ids_hbm, mask_hbm, tab_hbm, out_hbm, idx_v, mask_v, rows_a,
             rows_b, corr_v, acc_v, sem):
    c = lax.axis_index("c")
    s = lax.axis_index("s")
    w = s * NC + c
    i = w // 2
    h = w % 2

    ids_cp = pltpu.make_async_copy(ids_hbm.at[i], idx_v.at[pl.ds(0, L)], sem)
    ids_cp.start()
    pltpu.sync_copy(mask_hbm.at[i], mask_v)
    ids_cp.wait()

    for g in range(G):
        acc_v[pl.ds(g * LANES, LANES)] = jnp.zeros((LANES,), jnp.float32)

    def prep_body(t, cnt):
        m = mask_v[pl.ds(t * LANES, LANES)]
        return cnt + jnp.where(m == 1, 1, 0).astype(jnp.int32)

    cnt = lax.fori_loop(0, L // LANES, prep_body,
                        jnp.zeros((LANES,), jnp.int32), unroll=4)
    n = cnt[0]
    for t in range(1, LANES):
        n = n + cnt[t]
    for t in range(CK // 8):
        idx_v[pl.ds(L + t * 8, 8)] = jnp.zeros((8,), jnp.int32)

    T = (n + K - 1) // K

    def make_copy(j, buf):
        return pltpu.make_async_copy(
            tab_hbm.at[idx_v.at[pl.ds(j * K, K)], pl.ds(h * HALF, HALF)],
            buf, sem)

    def accum(buf):
        for g in range(G):
            gs = pl.ds(g * LANES, LANES)
            p = [buf[q, gs] for q in range(4)]
            for r in range(4, K, 4):
                for q in range(4):
                    p[q] = p[q] + buf[r + q, gs]
            plsc.addupdate(acc_v.at[gs], (p[0] + p[1]) + (p[2] + p[3]))

    @pl.when(T > 0)
    def _():
        make_copy(0, rows_a).start()

    def loop_body(j, _):
        even = lax.rem(j, 2) == 0
        nxt = j + 1 < T

        @pl.when(jnp.logical_and(nxt, even))
        def _():
            make_copy(j + 1, rows_b).start()

        @pl.when(jnp.logical_and(nxt, jnp.logical_not(even)))
        def _():
            make_copy(j + 1, rows_a).start()

        @pl.when(even)
        def _():
            make_copy(j, rows_a).wait()
            accum(rows_a)

        @pl.when(jnp.logical_not(even))
        def _():
            make_copy(j, rows_b).wait()
            accum(rows_b)

        return 0

    lax.fori_loop(0, T, loop_body, 0)

    @pl.when(T > 0)
    def _():
        start = (n // 8) * 8
        corr_cp = pltpu.make_async_copy(
            tab_hbm.at[idx_v.at[pl.ds(start, CK)], pl.ds(h * HALF, HALF)],
            corr_v, sem)
        corr_cp.start()
        corr_cp.wait()
        top = T * K
        for r in range(CK):
            pos = start + r
            sub = jnp.logical_and(pos >= n, pos < top)
            wgt = jnp.where(sub, -1.0, 0.0).astype(jnp.float32)
            for g in range(G):
                gs = pl.ds(g * LANES, LANES)
                plsc.addupdate(acc_v.at[gs], corr_v[r, gs] * wgt)

    nf = n.astype(jnp.float32)
    for g in range(G):
        acc_v[pl.ds(g * LANES, LANES)] = acc_v[pl.ds(g * LANES, LANES)] / nf
    pltpu.sync_copy(acc_v, out_hbm.at[w])


@jax.jit
def _sc_call(ids, mask, tab):
    mesh = plsc.VectorSubcoreMesh(core_axis_name="c", subcore_axis_name="s",
                                  num_cores=NC, num_subcores=NS)
    fn = pl.kernel(
        _sc_body,
        out_type=jax.ShapeDtypeStruct((NC * NS, HALF), jnp.float32),
        mesh=mesh,
        scratch_types=[
            pltpu.VMEM((L + CK,), jnp.int32),
            pltpu.VMEM((L,), jnp.int32),
            pltpu.VMEM((K, HALF), jnp.float32),
            pltpu.VMEM((K, HALF), jnp.float32),
            pltpu.VMEM((CK, HALF), jnp.float32),
            pltpu.VMEM((HALF,), jnp.float32),
            pltpu.SemaphoreType.DMA,
        ],
    )
    return fn(ids, mask, tab)


def kernel(tag_input_ids, tag_attention_mask, table):
    ids = tag_input_ids.astype(jnp.int32)
    mask = tag_attention_mask.astype(jnp.int32)
    out2 = _sc_call(ids, mask, table)
    return out2.reshape(B, DIM)


# speedup vs baseline: 4.1830x; 1.0212x over previous
import jax
import jax.numpy as jnp
from jax import lax
from jax.experimental import pallas as pl
from jax.experimental.pallas import tpu as pltpu
from jax.experimental.pallas import tpu_sc as plsc

B = 16
L = 2048
DIM = 768
HALF = DIM // 2
NC = 2
NS = 16
LANES = 16
K = 96
G = HALF // LANES
PADB = 8


def _sc_body(ids_hbm, mask_hbm, tab_hbm, out_hbm, idx_v, mask_v, rows_a,
             rows_b, corr_v, acc_v, sem, sem2):
    c = lax.axis_index("c")
    s = lax.axis_index("s")
    w = s * NC + c
    i = w // 2
    h = w % 2
    cols = pl.ds(h * HALF, HALF)

    ids_cp = pltpu.make_async_copy(ids_hbm.at[i], idx_v.at[pl.ds(0, L)], sem)
    ids_cp.start()
    pltpu.sync_copy(mask_hbm.at[i], mask_v)
    ids_cp.wait()

    corr_cp = pltpu.make_async_copy(
        tab_hbm.at[idx_v.at[pl.ds(0, 8)], cols], corr_v, sem2)
    corr_cp.start()

    for g in range(G):
        acc_v[pl.ds(g * LANES, LANES)] = jnp.zeros((LANES,), jnp.float32)

    def count_body(t, cnt):
        m = mask_v[pl.ds(t * LANES, LANES)]
        return cnt + jnp.where(m == 1, 1, 0).astype(jnp.int32)

    cnt = lax.fori_loop(0, L // LANES, count_body,
                        jnp.zeros((LANES,), jnp.int32), unroll=4)
    n = cnt[0]
    for t in range(1, LANES):
        n = n + cnt[t]

    T = (n + K - 1) // K

    head = idx_v[pl.ds(0, LANES)]
    idx0 = head[0]
    n16 = (n // LANES) * LANES
    iota = lax.iota(jnp.int32, LANES)

    def blend_body(jj, _):
        p = n16 + jj * LANES
        v = idx_v[pl.ds(p, LANES)]
        keep = (p + iota) < n
        idx_v[pl.ds(p, LANES)] = jnp.where(keep, v, idx0)
        return 0

    lax.fori_loop(0, PADB, blend_body, 0)

    def make_copy(j, buf):
        return pltpu.make_async_copy(
            tab_hbm.at[idx_v.at[pl.ds(j * K, K)], cols], buf, sem)

    def accum(buf):
        for g in range(G):
            gs = pl.ds(g * LANES, LANES)
            p = [buf[q, gs] for q in range(4)]
            for r in range(4, K, 4):
                for q in range(4):
                    p[q] = p[q] + buf[r + q, gs]
            plsc.addupdate(acc_v.at[gs], (p[0] + p[1]) + (p[2] + p[3]))

    @pl.when(T > 0)
    def _():
        make_copy(0, rows_a).start()

    def loop_body(j, _):
        even = lax.rem(j, 2) == 0
        nxt = j + 1 < T

        @pl.when(jnp.logical_and(nxt, even))
        def _():
            make_copy(j + 1, rows_b).start()

        @pl.when(jnp.logical_and(nxt, jnp.logical_not(even)))
        def _():
            make_copy(j + 1, rows_a).start()

        @pl.when(even)
        def _():
            make_copy(j, rows_a).wait()
            accum(rows_a)

        @pl.when(jnp.logical_not(even))
        def _():
            make_copy(j, rows_b).wait()
            accum(rows_b)

        return 0

    lax.fori_loop(0, T, loop_body, 0)

    corr_cp.wait()
    pad_f = (T * K - n).astype(jnp.float32)
    nf = n.astype(jnp.float32)
    for g in range(G):
        gs = pl.ds(g * LANES, LANES)
        acc_v[gs] = (acc_v[gs] - pad_f * corr_v[0, gs]) / nf
    pltpu.sync_copy(acc_v, out_hbm.at[w])


@jax.jit
def _sc_call(ids, mask, tab):
    mesh = plsc.VectorSubcoreMesh(core_axis_name="c", subcore_axis_name="s",
                                  num_cores=NC, num_subcores=NS)
    fn = pl.kernel(
        _sc_body,
        out_type=jax.ShapeDtypeStruct((NC * NS, HALF), jnp.float32),
        mesh=mesh,
        scratch_types=[
            pltpu.VMEM((L + PADB * LANES,), jnp.int32),
            pltpu.VMEM((L,), jnp.int32),
            pltpu.VMEM((K, HALF), jnp.float32),
            pltpu.VMEM((K, HALF), jnp.float32),
            pltpu.VMEM((8, HALF), jnp.float32),
            pltpu.VMEM((HALF,), jnp.float32),
            pltpu.SemaphoreType.DMA,
            pltpu.SemaphoreType.DMA,
        ],
    )
    return fn(ids, mask, tab)


def kernel(tag_input_ids, tag_attention_mask, table):
    ids = tag_input_ids.astype(jnp.int32)
    mask = tag_attention_mask.astype(jnp.int32)
    out2 = _sc_call(ids, mask, table)
    return out2.reshape(B, DIM)
